# baseline (speedup 1.0000x reference)
import functools

import jax
import jax.numpy as jnp
from jax import lax
from jax.experimental import pallas as pl
from jax.experimental.pallas import tpu as pltpu
from jax.experimental.pallas import tpu_sc as plsc

N = 10000
NP = 10240
F = 256
H = 256
O = 128
E = 160000
EP = 163840
NC = 2
NS = 16
BM = 256
GB = NP // BM
PER_SUB = EP // NS
PER_W = EP // (NC * NS)
CH = 128
CHG = 80
ROWS_SUB = NP // NS
def _dot(a, b):
    return jnp.dot(a.astype(jnp.bfloat16), b.astype(jnp.bfloat16),
                   preferred_element_type=jnp.float32)


def _mesh():
    return plsc.VectorSubcoreMesh(
        core_axis_name="c", subcore_axis_name="s", num_cores=NC, num_subcores=NS)


def _layer1_body(x_ref, agg_ref, wr_ref, wl_ref, b_ref, h_ref):
    aggb = jnp.concatenate([agg_ref[0], agg_ref[1]], axis=1)
    h = _dot(x_ref[...], wr_ref[...]) + _dot(aggb, wl_ref[...]) + b_ref[...]
    h = jnp.maximum(h, 0.0)
    h_ref[...] = jnp.stack([h[:, :128], h[:, 128:]], axis=0)


def _layer1(x, agg, wroot, wrel, b):
    return pl.pallas_call(
        _layer1_body,
        grid=(GB,),
        in_specs=[pl.BlockSpec((BM, F), lambda i: (i, 0)),
                  pl.BlockSpec((2, BM, 128), lambda i: (0, i, 0)),
                  pl.BlockSpec((F, H), lambda i: (0, 0)),
                  pl.BlockSpec((F, H), lambda i: (0, 0)),
                  pl.BlockSpec((1, H), lambda i: (0, 0))],
        out_specs=pl.BlockSpec((2, BM, 128), lambda i: (0, i, 0)),
        out_shape=jax.ShapeDtypeStruct((2, NP, 128), jnp.float32),
    )(x, agg, wroot, wrel, b)


def _attn_body(h1_ref, agg_ref, wr_ref, wl_ref, b_ref, wa_ref, as_ref, ad_ref,
               ha_ref, sa_ref, da_ref):
    h1b = jnp.concatenate([h1_ref[0], h1_ref[1]], axis=1)
    aggb = jnp.concatenate([agg_ref[0], agg_ref[1]], axis=1)
    h = _dot(h1b, wr_ref[...]) + _dot(aggb, wl_ref[...]) + b_ref[...]
    h = jnp.maximum(h, 0.0)
    ha = _dot(h, wa_ref[...])
    ha_ref[...] = jnp.stack([ha[:, :128], ha[:, 128:]], axis=0)
    sa_ref[...] = _dot(ha, as_ref[...])
    da_ref[...] = _dot(ha, ad_ref[...])


def _attn(h1t, agg, wroot, wrel, b, wa, a_src, a_dst):
    return pl.pallas_call(
        _attn_body,
        grid=(GB,),
        in_specs=[pl.BlockSpec((2, BM, 128), lambda i: (0, i, 0)),
                  pl.BlockSpec((2, BM, 128), lambda i: (0, i, 0)),
                  pl.BlockSpec((H, H), lambda i: (0, 0)),
                  pl.BlockSpec((H, H), lambda i: (0, 0)),
                  pl.BlockSpec((1, H), lambda i: (0, 0)),
                  pl.BlockSpec((H, H), lambda i: (0, 0)),
                  pl.BlockSpec((H, 1), lambda i: (0, 0)),
                  pl.BlockSpec((H, 1), lambda i: (0, 0))],
        out_specs=[pl.BlockSpec((2, BM, 128), lambda i: (0, i, 0)),
                   pl.BlockSpec((BM, 1), lambda i: (i, 0)),
                   pl.BlockSpec((BM, 1), lambda i: (i, 0))],
        out_shape=[jax.ShapeDtypeStruct((2, NP, 128), jnp.float32),
                   jax.ShapeDtypeStruct((NP, 1), jnp.float32),
                   jax.ShapeDtypeStruct((NP, 1), jnp.float32)],
    )(h1t, agg, wroot, wrel, b, wa, a_src, a_dst)


def _fin_body(g_ref, d_ref, w_ref, b_ref, o_ref):
    h = jnp.concatenate([g_ref[0], g_ref[1]], axis=1)
    h = h / (d_ref[...] + 1e-16)
    o_ref[...] = _dot(h, w_ref[...]) + b_ref[...]


def _final(gat, denom, wfc, bfc):
    return pl.pallas_call(
        _fin_body,
        grid=(GB,),
        in_specs=[pl.BlockSpec((2, BM, 128), lambda i: (0, i, 0)),
                  pl.BlockSpec((BM, 1), lambda i: (i, 0)),
                  pl.BlockSpec((H, O), lambda i: (0, 0)),
                  pl.BlockSpec((1, O), lambda i: (0, 0))],
        out_specs=pl.BlockSpec((BM, O), lambda i: (i, 0)),
        out_shape=jax.ShapeDtypeStruct((NP, O), jnp.float32),
    )(gat, denom, wfc, bfc)


def _seg_body(tab, srcp, dstp, out, idx_s, idx_g, idxd0, idxd1,
              rows0, rows1, acc, gsem, ssem0, ssem1):
    c = lax.axis_index("c")
    s = lax.axis_index("s")
    zero16 = jnp.zeros((16,), jnp.float32)

    def zrow(i, carry):
        for j in range(8):
            rows0[i, pl.ds(j * 16, 16)] = zero16
        return carry
    lax.fori_loop(0, CH, zrow, 0)

    def zacc(k, carry):
        pltpu.sync_copy(rows0, acc.at[pl.ds(s * ROWS_SUB + k * CH, CH)])
        return carry
    lax.fori_loop(0, ROWS_SUB // CH, zacc, 0)
    plsc.subcore_barrier()

    base0 = s * PER_SUB
    coff = c * NP

    def step(k, par, rw, ix, sm):
        i = 2 * k + par
        b = base0 + i * CH

        @pl.when(k >= 1)
        def _():
            pltpu.make_async_copy(tab.at[pl.ds(0, CH)], rw, sm).wait()
        pltpu.sync_copy(srcp.at[pl.ds(b, CH)], idx_s)
        pltpu.sync_copy(dstp.at[pl.ds(b, CH)], ix)
        for j in range(CH // 16):
            idx_g[pl.ds(j * 16, 16)] = idx_s[pl.ds(j * 16, 16)] + coff
        pltpu.async_copy(tab.at[idx_g], rw, gsem).wait()
        pltpu.async_copy(rw, acc.at[ix], sm, add=True)

    def pair(k, carry):
        step(k, 0, rows0, idxd0, ssem0)
        step(k, 1, rows1, idxd1, ssem1)
        return carry
    lax.fori_loop(0, PER_SUB // CH // 2, pair, 0)
    pltpu.make_async_copy(tab.at[pl.ds(0, CH)], rows0, ssem0).wait()
    pltpu.make_async_copy(tab.at[pl.ds(0, CH)], rows1, ssem1).wait()
    plsc.subcore_barrier()

    r0 = s * ROWS_SUB

    @pl.when(c == 0)
    def _():
        pltpu.sync_copy(acc.at[pl.ds(r0, ROWS_SUB)], out.at[0, pl.ds(r0, ROWS_SUB)])

    @pl.when(c == 1)
    def _():
        pltpu.sync_copy(acc.at[pl.ds(r0, ROWS_SUB)], out.at[1, pl.ds(r0, ROWS_SUB)])


def _seg_sum(tab, srcp, dstp):
    return pl.kernel(
        _seg_body,
        out_type=jax.ShapeDtypeStruct((2, NP, 128), jnp.float32),
        mesh=_mesh(),
        compiler_params=pltpu.CompilerParams(needs_layout_passes=False),
        scratch_types=[
            pltpu.VMEM((CH,), jnp.int32),
            pltpu.VMEM((CH,), jnp.int32),
            pltpu.VMEM((CH,), jnp.int32),
            pltpu.VMEM((CH,), jnp.int32),
            pltpu.VMEM((CH, 128), jnp.float32),
            pltpu.VMEM((CH, 128), jnp.float32),
            pltpu.VMEM_SHARED((NP, 128), jnp.float32),
            pltpu.SemaphoreType.DMA,
            pltpu.SemaphoreType.DMA,
            pltpu.SemaphoreType.DMA,
        ],
    )(tab, srcp, dstp)


def _attns_body(asrc, adst, srcp, dstp, exo, dno,
                av, bv, sb, db, eb, macc, dacc, rbuf, mred, sacc):
    c = lax.axis_index("c")
    s = lax.axis_index("s")
    pltpu.sync_copy(asrc, av)
    pltpu.sync_copy(adst, bv)
    e0 = s * PER_SUB
    pltpu.sync_copy(srcp.at[pl.ds(e0, PER_SUB)], sb)
    pltpu.sync_copy(dstp.at[pl.ds(e0, PER_SUB)], db)
    ninf = jnp.full((16,), -jnp.inf, jnp.float32)

    def initm(i, carry):
        macc[pl.ds(i * 16, 16)] = ninf
        return carry
    lax.fori_loop(0, NP // 16, initm, 0)

    def edge_grp(i, carry):
        s16 = sb[pl.ds(i * 16, 16)]
        d16 = db[pl.ds(i * 16, 16)]
        t = plsc.load_gather(av, [s16]) + plsc.load_gather(bv, [d16])
        e16 = jnp.where(t > 0, t, t * 0.2)
        eb[pl.ds(i * 16, 16)] = e16
        cur = plsc.load_gather(macc, [d16])
        m0 = e16 > cur

        def wcond(carry_w):
            return jnp.max(carry_w[2].astype(jnp.int32)) > 0

        def wbody(carry_w):
            e_, d_, m_ = carry_w
            plsc.store_scatter(macc, [d_], e_, mask=m_)
            cur2 = plsc.load_gather(macc, [d_])
            return (e_, d_, m_ & (e_ > cur2))
        lax.while_loop(wcond, wbody, (e16, d16, m0))
        return carry
    lax.fori_loop(0, PER_SUB // 16, edge_grp, 0)

    pltpu.sync_copy(macc, sacc.at[s])
    plsc.subcore_barrier()
    r0 = s * ROWS_SUB
    pltpu.sync_copy(sacc.at[0, pl.ds(r0, ROWS_SUB)], mred)

    def redt(t_, carry):
        pltpu.sync_copy(sacc.at[t_, pl.ds(r0, ROWS_SUB)], rbuf)

        def mx(i, carry2):
            sl = pl.ds(i * 16, 16)
            mred[sl] = jnp.maximum(mred[sl], rbuf[sl])
            return carry2
        lax.fori_loop(0, ROWS_SUB // 16, mx, 0)
        return carry
    lax.fori_loop(1, NS, redt, 0)
    pltpu.sync_copy(mred, sacc.at[0, pl.ds(r0, ROWS_SUB)])
    plsc.subcore_barrier()
    pltpu.sync_copy(sacc.at[0], av)
    zero16 = jnp.zeros((16,), jnp.float32)

    def initd(i, carry):
        dacc[pl.ds(i * 16, 16)] = zero16
        return carry
    lax.fori_loop(0, NP // 16, initd, 0)

    iota16 = lax.iota(jnp.int32, 16)
    dn = lax.GatherDimensionNumbers(
        offset_dims=(), collapsed_slice_dims=(0,), start_index_map=(0,))

    def _take(v, i):
        return lax.gather(v, i[:, None], dn, slice_sizes=(1,),
                          mode=lax.GatherScatterMode.PROMISE_IN_BOUNDS)

    def p2(i, carry):
        sl = pl.ds(i * 16, 16)
        d16 = db[sl]
        mx16 = plsc.load_gather(av, [d16])
        x16 = jnp.exp(eb[sl] - mx16)
        eb[sl] = x16
        dk, xv = plsc.sort_key_val(d16, x16)
        for k in (1, 2, 4, 8):
            sh = jnp.maximum(iota16 - k, 0)
            xs = _take(xv, sh)
            ds_ = _take(dk, sh)
            xv = xv + jnp.where((iota16 >= k) & (ds_ == dk), xs, 0.0)
        nxt = _take(dk, jnp.minimum(iota16 + 1, 15))
        last = (iota16 == 15) | (nxt != dk)
        cur = plsc.load_gather(dacc, [dk])
        plsc.store_scatter(dacc, [dk], cur + xv, mask=last)
        return carry
    lax.fori_loop(0, PER_SUB // 16, p2, 0)
    off = c * PER_W
    pltpu.sync_copy(eb.at[pl.ds(off, PER_W)], exo.at[pl.ds(e0 + off, PER_W)])

    pltpu.sync_copy(dacc, sacc.at[s])
    plsc.subcore_barrier()
    pltpu.sync_copy(sacc.at[0, pl.ds(r0, ROWS_SUB)], mred)

    def redt2(t_, carry):
        pltpu.sync_copy(sacc.at[t_, pl.ds(r0, ROWS_SUB)], rbuf)

        def sm(i, carry2):
            sl = pl.ds(i * 16, 16)
            mred[sl] = mred[sl] + rbuf[sl]
            return carry2
        lax.fori_loop(0, ROWS_SUB // 16, sm, 0)
        return carry
    lax.fori_loop(1, NS, redt2, 0)

    @pl.when(c == 0)
    def _():
        pltpu.sync_copy(mred, dno.at[pl.ds(r0, ROWS_SUB)])


def _attn_scalar(asrc, adst, srcp, dstp):
    return pl.kernel(
        _attns_body,
        out_type=[jax.ShapeDtypeStruct((EP,), jnp.float32),
                  jax.ShapeDtypeStruct((NP,), jnp.float32)],
        mesh=_mesh(),
        compiler_params=pltpu.CompilerParams(needs_layout_passes=False),
        scratch_types=[
            pltpu.VMEM((NP,), jnp.float32),
            pltpu.VMEM((NP,), jnp.float32),
            pltpu.VMEM((PER_SUB,), jnp.int32),
            pltpu.VMEM((PER_SUB,), jnp.int32),
            pltpu.VMEM((PER_SUB,), jnp.float32),
            pltpu.VMEM((NP,), jnp.float32),
            pltpu.VMEM((NP,), jnp.float32),
            pltpu.VMEM((ROWS_SUB,), jnp.float32),
            pltpu.VMEM((ROWS_SUB,), jnp.float32),
            pltpu.VMEM_SHARED((NS, NP), jnp.float32),
        ],
    )(asrc, adst, srcp, dstp)


def _gat_body(tab, srcp, dstp, exv, out,
              idx_s, idx_g, idxd0, idxd1, wv, rows, rows20, rows21,
              acc, gsem, ssem0, ssem1):
    c = lax.axis_index("c")
    s = lax.axis_index("s")
    zero16 = jnp.zeros((16,), jnp.float32)

    def zrow(i, carry):
        for j in range(8):
            rows20[i, pl.ds(j * 16, 16)] = zero16
        return carry
    lax.fori_loop(0, CHG, zrow, 0)

    def zacc(k, carry):
        pltpu.sync_copy(rows20, acc.at[pl.ds(s * ROWS_SUB + k * CHG, CHG)])
        return carry
    lax.fori_loop(0, ROWS_SUB // CHG, zacc, 0)
    plsc.subcore_barrier()

    base0 = s * PER_SUB
    coff = c * NP
    dn = lax.GatherDimensionNumbers(
        offset_dims=(), collapsed_slice_dims=(0,), start_index_map=(0,))

    def _take(v, i):
        return lax.gather(v, i[:, None], dn, slice_sizes=(1,),
                          mode=lax.GatherScatterMode.PROMISE_IN_BOUNDS)

    def step(k, par, r2, ix, sm):
        i = 2 * k + par
        b = base0 + i * CHG

        @pl.when(k >= 1)
        def _():
            pltpu.make_async_copy(tab.at[pl.ds(0, CHG)], r2, sm).wait()
        pltpu.sync_copy(srcp.at[pl.ds(b, CHG)], idx_s)
        pltpu.sync_copy(dstp.at[pl.ds(b, CHG)], ix)
        pltpu.sync_copy(exv.at[pl.ds(b, CHG)], wv)
        for j in range(CHG // 16):
            idx_g[pl.ds(j * 16, 16)] = idx_s[pl.ds(j * 16, 16)] + coff
        pltpu.async_copy(tab.at[idx_g], rows, gsem).wait()

        def grp(g, carry2):
            w16 = wv[pl.ds(g * 16, 16)]
            for l in range(16):
                wb = _take(w16, jnp.full((16,), l, jnp.int32))
                e = g * 16 + l
                for j in range(8):
                    sl = pl.ds(j * 16, 16)
                    r2[e, sl] = rows[e, sl] * wb
            return carry2
        lax.fori_loop(0, CHG // 16, grp, 0)
        pltpu.async_copy(r2, acc.at[ix], sm, add=True)

    def pair(k, carry):
        step(k, 0, rows20, idxd0, ssem0)
        step(k, 1, rows21, idxd1, ssem1)
        return carry
    lax.fori_loop(0, PER_SUB // CHG // 2, pair, 0)
    pltpu.make_async_copy(tab.at[pl.ds(0, CHG)], rows20, ssem0).wait()
    pltpu.make_async_copy(tab.at[pl.ds(0, CHG)], rows21, ssem1).wait()
    plsc.subcore_barrier()

    r0 = s * ROWS_SUB

    @pl.when(c == 0)
    def _():
        pltpu.sync_copy(acc.at[pl.ds(r0, ROWS_SUB)], out.at[0, pl.ds(r0, ROWS_SUB)])

    @pl.when(c == 1)
    def _():
        pltpu.sync_copy(acc.at[pl.ds(r0, ROWS_SUB)], out.at[1, pl.ds(r0, ROWS_SUB)])


def _gat(tab, srcp, dstp, exv):
    return pl.kernel(
        _gat_body,
        out_type=jax.ShapeDtypeStruct((2, NP, 128), jnp.float32),
        mesh=_mesh(),
        compiler_params=pltpu.CompilerParams(needs_layout_passes=False),
        scratch_types=[
            pltpu.VMEM((CHG,), jnp.int32),
            pltpu.VMEM((CHG,), jnp.int32),
            pltpu.VMEM((CHG,), jnp.int32),
            pltpu.VMEM((CHG,), jnp.int32),
            pltpu.VMEM((CHG,), jnp.float32),
            pltpu.VMEM((CHG, 128), jnp.float32),
            pltpu.VMEM((CHG, 128), jnp.float32),
            pltpu.VMEM((CHG, 128), jnp.float32),
            pltpu.VMEM_SHARED((NP, 128), jnp.float32),
            pltpu.SemaphoreType.DMA,
            pltpu.SemaphoreType.DMA,
            pltpu.SemaphoreType.DMA,
        ],
    )(tab, srcp, dstp, exv)


def kernel(x, edge_index, W1_root, W1_rel, b1, W2_root, W2_rel, b2,
           Wa, a_src, a_dst, Wfc, bfc):
    f32 = jnp.float32
    xp = jnp.zeros((NP, F), f32).at[:N].set(x)
    pad = jnp.full((EP - E,), NP - 1, jnp.int32)
    srcp = jnp.concatenate([edge_index[0].astype(jnp.int32), pad])
    dstp = jnp.concatenate([edge_index[1].astype(jnp.int32), pad])

    xt = xp.reshape(NP, 2, 128).transpose(1, 0, 2).reshape(2 * NP, 128)
    agg1 = _seg_sum(xt, srcp, dstp)
    h1t = _layer1(xp, agg1, W1_root, W1_rel, b1.reshape(1, H))
    agg2 = _seg_sum(h1t.reshape(2 * NP, 128), srcp, dstp)
    ha, asv, adv = _attn(h1t, agg2, W2_root, W2_rel, b2.reshape(1, H),
                         Wa, a_src.reshape(H, 1), a_dst.reshape(H, 1))
    ex, denom = _attn_scalar(asv.reshape(NP), adv.reshape(NP), srcp, dstp)
    gat = _gat(ha.reshape(2 * NP, 128), srcp, dstp, ex)
    out = _final(gat, denom.reshape(NP, 1), Wfc, bfc.reshape(1, O))
    return out[:N]


# speedup vs baseline: 1.0164x; 1.0164x over previous
import functools

import jax
import jax.numpy as jnp
from jax import lax
from jax.experimental import pallas as pl
from jax.experimental.pallas import tpu as pltpu
from jax.experimental.pallas import tpu_sc as plsc

N = 10000
NP = 10240
F = 256
H = 256
O = 128
E = 160000
EP = 163840
NC = 2
NS = 16
BM = 256
GB = NP // BM
PER_SUB = EP // NS
PER_W = EP // (NC * NS)
CH = 128
ROWS_SUB = NP // NS
def _dot(a, b):
    return jnp.dot(a.astype(jnp.bfloat16), b.astype(jnp.bfloat16),
                   preferred_element_type=jnp.float32)


def _mesh():
    return plsc.VectorSubcoreMesh(
        core_axis_name="c", subcore_axis_name="s", num_cores=NC, num_subcores=NS)


def _layer1_body(x_ref, agg_ref, wr_ref, wl_ref, b_ref, h_ref):
    aggb = jnp.concatenate([agg_ref[0], agg_ref[1]], axis=1)
    h = _dot(x_ref[...], wr_ref[...]) + _dot(aggb, wl_ref[...]) + b_ref[...]
    h = jnp.maximum(h, 0.0)
    h_ref[...] = jnp.stack([h[:, :128], h[:, 128:]], axis=0)


def _layer1(x, agg, wroot, wrel, b):
    return pl.pallas_call(
        _layer1_body,
        grid=(GB,),
        in_specs=[pl.BlockSpec((BM, F), lambda i: (i, 0)),
                  pl.BlockSpec((2, BM, 128), lambda i: (0, i, 0)),
                  pl.BlockSpec((F, H), lambda i: (0, 0)),
                  pl.BlockSpec((F, H), lambda i: (0, 0)),
                  pl.BlockSpec((1, H), lambda i: (0, 0))],
        out_specs=pl.BlockSpec((2, BM, 128), lambda i: (0, i, 0)),
        out_shape=jax.ShapeDtypeStruct((2, NP, 128), jnp.float32),
    )(x, agg, wroot, wrel, b)


def _attn_body(h1_ref, agg_ref, wr_ref, wl_ref, b_ref, wa_ref, as_ref, ad_ref,
               ha_ref, sa_ref, da_ref):
    h1b = jnp.concatenate([h1_ref[0], h1_ref[1]], axis=1)
    aggb = jnp.concatenate([agg_ref[0], agg_ref[1]], axis=1)
    h = _dot(h1b, wr_ref[...]) + _dot(aggb, wl_ref[...]) + b_ref[...]
    h = jnp.maximum(h, 0.0)
    ha = _dot(h, wa_ref[...])
    ha_ref[...] = jnp.stack([ha[:, :128], ha[:, 128:]], axis=0)
    sa_ref[...] = _dot(ha, as_ref[...])
    da_ref[...] = _dot(ha, ad_ref[...])


def _attn(h1t, agg, wroot, wrel, b, wa, a_src, a_dst):
    return pl.pallas_call(
        _attn_body,
        grid=(GB,),
        in_specs=[pl.BlockSpec((2, BM, 128), lambda i: (0, i, 0)),
                  pl.BlockSpec((2, BM, 128), lambda i: (0, i, 0)),
                  pl.BlockSpec((H, H), lambda i: (0, 0)),
                  pl.BlockSpec((H, H), lambda i: (0, 0)),
                  pl.BlockSpec((1, H), lambda i: (0, 0)),
                  pl.BlockSpec((H, H), lambda i: (0, 0)),
                  pl.BlockSpec((H, 1), lambda i: (0, 0)),
                  pl.BlockSpec((H, 1), lambda i: (0, 0))],
        out_specs=[pl.BlockSpec((2, BM, 128), lambda i: (0, i, 0)),
                   pl.BlockSpec((BM, 1), lambda i: (i, 0)),
                   pl.BlockSpec((BM, 1), lambda i: (i, 0))],
        out_shape=[jax.ShapeDtypeStruct((2, NP, 128), jnp.float32),
                   jax.ShapeDtypeStruct((NP, 1), jnp.float32),
                   jax.ShapeDtypeStruct((NP, 1), jnp.float32)],
    )(h1t, agg, wroot, wrel, b, wa, a_src, a_dst)


def _fin_body(g_ref, d_ref, w_ref, b_ref, o_ref):
    h = jnp.concatenate([g_ref[0], g_ref[1]], axis=1)
    h = h / (d_ref[...] + 1e-16)
    o_ref[...] = _dot(h, w_ref[...]) + b_ref[...]


def _final(gat, denom, wfc, bfc):
    return pl.pallas_call(
        _fin_body,
        grid=(GB,),
        in_specs=[pl.BlockSpec((2, BM, 128), lambda i: (0, i, 0)),
                  pl.BlockSpec((BM, 1), lambda i: (i, 0)),
                  pl.BlockSpec((H, O), lambda i: (0, 0)),
                  pl.BlockSpec((1, O), lambda i: (0, 0))],
        out_specs=pl.BlockSpec((BM, O), lambda i: (i, 0)),
        out_shape=jax.ShapeDtypeStruct((NP, O), jnp.float32),
    )(gat, denom, wfc, bfc)


def _seg_body(tab, srcp, dstp, out, idx_s, idx_g, idxd0, idxd1,
              rows0, rows1, acc, gsem, ssem0, ssem1):
    c = lax.axis_index("c")
    s = lax.axis_index("s")
    zero16 = jnp.zeros((16,), jnp.float32)

    def zrow(i, carry):
        for j in range(8):
            rows0[i, pl.ds(j * 16, 16)] = zero16
        return carry
    lax.fori_loop(0, CH, zrow, 0)

    def zacc(k, carry):
        pltpu.sync_copy(rows0, acc.at[pl.ds(s * ROWS_SUB + k * CH, CH)])
        return carry
    lax.fori_loop(0, ROWS_SUB // CH, zacc, 0)
    plsc.subcore_barrier()

    base0 = s * PER_SUB
    coff = c * NP

    def step(k, par, rw, ix, sm):
        i = 2 * k + par
        b = base0 + i * CH

        @pl.when(k >= 1)
        def _():
            pltpu.make_async_copy(tab.at[pl.ds(0, CH)], rw, sm).wait()
        pltpu.sync_copy(srcp.at[pl.ds(b, CH)], idx_s)
        pltpu.sync_copy(dstp.at[pl.ds(b, CH)], ix)
        for j in range(CH // 16):
            idx_g[pl.ds(j * 16, 16)] = idx_s[pl.ds(j * 16, 16)] + coff
        pltpu.async_copy(tab.at[idx_g], rw, gsem).wait()
        pltpu.async_copy(rw, acc.at[ix], sm, add=True)

    def pair(k, carry):
        step(k, 0, rows0, idxd0, ssem0)
        step(k, 1, rows1, idxd1, ssem1)
        return carry
    lax.fori_loop(0, PER_SUB // CH // 2, pair, 0)
    pltpu.make_async_copy(tab.at[pl.ds(0, CH)], rows0, ssem0).wait()
    pltpu.make_async_copy(tab.at[pl.ds(0, CH)], rows1, ssem1).wait()
    plsc.subcore_barrier()

    r0 = s * ROWS_SUB

    @pl.when(c == 0)
    def _():
        pltpu.sync_copy(acc.at[pl.ds(r0, ROWS_SUB)], out.at[0, pl.ds(r0, ROWS_SUB)])

    @pl.when(c == 1)
    def _():
        pltpu.sync_copy(acc.at[pl.ds(r0, ROWS_SUB)], out.at[1, pl.ds(r0, ROWS_SUB)])


def _seg_sum(tab, srcp, dstp):
    return pl.kernel(
        _seg_body,
        out_type=jax.ShapeDtypeStruct((2, NP, 128), jnp.float32),
        mesh=_mesh(),
        compiler_params=pltpu.CompilerParams(needs_layout_passes=False),
        scratch_types=[
            pltpu.VMEM((CH,), jnp.int32),
            pltpu.VMEM((CH,), jnp.int32),
            pltpu.VMEM((CH,), jnp.int32),
            pltpu.VMEM((CH,), jnp.int32),
            pltpu.VMEM((CH, 128), jnp.float32),
            pltpu.VMEM((CH, 128), jnp.float32),
            pltpu.VMEM_SHARED((NP, 128), jnp.float32),
            pltpu.SemaphoreType.DMA,
            pltpu.SemaphoreType.DMA,
            pltpu.SemaphoreType.DMA,
        ],
    )(tab, srcp, dstp)


def _attns_body(asrc, adst, srcp, dstp, exo, dno,
                av, bv, sb, db, eb, macc, dacc, rbuf, mred, sacc):
    c = lax.axis_index("c")
    s = lax.axis_index("s")
    pltpu.sync_copy(asrc, av)
    pltpu.sync_copy(adst, bv)
    e0 = s * PER_SUB
    pltpu.sync_copy(srcp.at[pl.ds(e0, PER_SUB)], sb)
    pltpu.sync_copy(dstp.at[pl.ds(e0, PER_SUB)], db)
    ninf = jnp.full((16,), -jnp.inf, jnp.float32)

    def initm(i, carry):
        macc[pl.ds(i * 16, 16)] = ninf
        return carry
    lax.fori_loop(0, NP // 16, initm, 0)

    def edge_grp(i, carry):
        s16 = sb[pl.ds(i * 16, 16)]
        d16 = db[pl.ds(i * 16, 16)]
        t = plsc.load_gather(av, [s16]) + plsc.load_gather(bv, [d16])
        e16 = jnp.where(t > 0, t, t * 0.2)
        eb[pl.ds(i * 16, 16)] = e16
        cur = plsc.load_gather(macc, [d16])
        m0 = e16 > cur

        def wcond(carry_w):
            return jnp.max(carry_w[2].astype(jnp.int32)) > 0

        def wbody(carry_w):
            e_, d_, m_ = carry_w
            plsc.store_scatter(macc, [d_], e_, mask=m_)
            cur2 = plsc.load_gather(macc, [d_])
            return (e_, d_, m_ & (e_ > cur2))
        lax.while_loop(wcond, wbody, (e16, d16, m0))
        return carry
    lax.fori_loop(0, PER_SUB // 16, edge_grp, 0)

    pltpu.sync_copy(macc, sacc.at[s])
    plsc.subcore_barrier()
    r0 = s * ROWS_SUB
    pltpu.sync_copy(sacc.at[0, pl.ds(r0, ROWS_SUB)], mred)

    def redt(t_, carry):
        pltpu.sync_copy(sacc.at[t_, pl.ds(r0, ROWS_SUB)], rbuf)

        def mx(i, carry2):
            sl = pl.ds(i * 16, 16)
            mred[sl] = jnp.maximum(mred[sl], rbuf[sl])
            return carry2
        lax.fori_loop(0, ROWS_SUB // 16, mx, 0)
        return carry
    lax.fori_loop(1, NS, redt, 0)
    pltpu.sync_copy(mred, sacc.at[0, pl.ds(r0, ROWS_SUB)])
    plsc.subcore_barrier()
    pltpu.sync_copy(sacc.at[0], av)
    zero16 = jnp.zeros((16,), jnp.float32)

    def initd(i, carry):
        dacc[pl.ds(i * 16, 16)] = zero16
        return carry
    lax.fori_loop(0, NP // 16, initd, 0)

    iota16 = lax.iota(jnp.int32, 16)
    dn = lax.GatherDimensionNumbers(
        offset_dims=(), collapsed_slice_dims=(0,), start_index_map=(0,))

    def _take(v, i):
        return lax.gather(v, i[:, None], dn, slice_sizes=(1,),
                          mode=lax.GatherScatterMode.PROMISE_IN_BOUNDS)

    def p2(i, carry):
        sl = pl.ds(i * 16, 16)
        d16 = db[sl]
        mx16 = plsc.load_gather(av, [d16])
        x16 = jnp.exp(eb[sl] - mx16)
        eb[sl] = x16
        dk, xv = plsc.sort_key_val(d16, x16)
        for k in (1, 2, 4, 8):
            sh = jnp.maximum(iota16 - k, 0)
            xs = _take(xv, sh)
            ds_ = _take(dk, sh)
            xv = xv + jnp.where((iota16 >= k) & (ds_ == dk), xs, 0.0)
        nxt = _take(dk, jnp.minimum(iota16 + 1, 15))
        last = (iota16 == 15) | (nxt != dk)
        cur = plsc.load_gather(dacc, [dk])
        plsc.store_scatter(dacc, [dk], cur + xv, mask=last)
        return carry
    lax.fori_loop(0, PER_SUB // 16, p2, 0)
    off = c * PER_W
    pltpu.sync_copy(eb.at[pl.ds(off, PER_W)], exo.at[pl.ds(e0 + off, PER_W)])

    pltpu.sync_copy(dacc, sacc.at[s])
    plsc.subcore_barrier()
    pltpu.sync_copy(sacc.at[0, pl.ds(r0, ROWS_SUB)], mred)

    def redt2(t_, carry):
        pltpu.sync_copy(sacc.at[t_, pl.ds(r0, ROWS_SUB)], rbuf)

        def sm(i, carry2):
            sl = pl.ds(i * 16, 16)
            mred[sl] = mred[sl] + rbuf[sl]
            return carry2
        lax.fori_loop(0, ROWS_SUB // 16, sm, 0)
        return carry
    lax.fori_loop(1, NS, redt2, 0)

    @pl.when(c == 0)
    def _():
        pltpu.sync_copy(mred, dno.at[pl.ds(r0, ROWS_SUB)])


def _attn_scalar(asrc, adst, srcp, dstp):
    return pl.kernel(
        _attns_body,
        out_type=[jax.ShapeDtypeStruct((EP,), jnp.float32),
                  jax.ShapeDtypeStruct((NP,), jnp.float32)],
        mesh=_mesh(),
        compiler_params=pltpu.CompilerParams(needs_layout_passes=False),
        scratch_types=[
            pltpu.VMEM((NP,), jnp.float32),
            pltpu.VMEM((NP,), jnp.float32),
            pltpu.VMEM((PER_SUB,), jnp.int32),
            pltpu.VMEM((PER_SUB,), jnp.int32),
            pltpu.VMEM((PER_SUB,), jnp.float32),
            pltpu.VMEM((NP,), jnp.float32),
            pltpu.VMEM((NP,), jnp.float32),
            pltpu.VMEM((ROWS_SUB,), jnp.float32),
            pltpu.VMEM((ROWS_SUB,), jnp.float32),
            pltpu.VMEM_SHARED((NS, NP), jnp.float32),
        ],
    )(asrc, adst, srcp, dstp)


def _gat_body(tab, srcp, dstp, exv, out,
              idx_s, idx_g, idx_d, wv, rows, rows2, acc, gsem):
    c = lax.axis_index("c")
    s = lax.axis_index("s")
    zero16 = jnp.zeros((16,), jnp.float32)

    def zrow(i, carry):
        for j in range(8):
            rows2[i, pl.ds(j * 16, 16)] = zero16
        return carry
    lax.fori_loop(0, CH, zrow, 0)

    def zacc(k, carry):
        pltpu.sync_copy(rows2, acc.at[pl.ds(s * ROWS_SUB + k * CH, CH)])
        return carry
    lax.fori_loop(0, ROWS_SUB // CH, zacc, 0)
    plsc.subcore_barrier()

    base0 = s * PER_SUB
    coff = c * NP
    dn = lax.GatherDimensionNumbers(
        offset_dims=(), collapsed_slice_dims=(0,), start_index_map=(0,))

    def _take(v, i):
        return lax.gather(v, i[:, None], dn, slice_sizes=(1,),
                          mode=lax.GatherScatterMode.PROMISE_IN_BOUNDS)

    def chunk(i, carry):
        b = base0 + i * CH
        pltpu.sync_copy(srcp.at[pl.ds(b, CH)], idx_s)
        pltpu.sync_copy(dstp.at[pl.ds(b, CH)], idx_d)
        pltpu.sync_copy(exv.at[pl.ds(b, CH)], wv)
        for j in range(CH // 16):
            idx_g[pl.ds(j * 16, 16)] = idx_s[pl.ds(j * 16, 16)] + coff
        pltpu.async_copy(tab.at[idx_g], rows, gsem).wait()

        def grp(g, carry2):
            w16 = wv[pl.ds(g * 16, 16)]
            for l in range(16):
                wb = _take(w16, jnp.full((16,), l, jnp.int32))
                e = g * 16 + l
                for j in range(8):
                    sl = pl.ds(j * 16, 16)
                    rows2[e, sl] = rows[e, sl] * wb
            return carry2
        lax.fori_loop(0, CH // 16, grp, 0)
        pltpu.sync_copy(rows2, acc.at[idx_d], add=True)
        return carry
    lax.fori_loop(0, PER_SUB // CH, chunk, 0)
    plsc.subcore_barrier()

    r0 = s * ROWS_SUB

    @pl.when(c == 0)
    def _():
        pltpu.sync_copy(acc.at[pl.ds(r0, ROWS_SUB)], out.at[0, pl.ds(r0, ROWS_SUB)])

    @pl.when(c == 1)
    def _():
        pltpu.sync_copy(acc.at[pl.ds(r0, ROWS_SUB)], out.at[1, pl.ds(r0, ROWS_SUB)])


def _gat(tab, srcp, dstp, exv):
    return pl.kernel(
        _gat_body,
        out_type=jax.ShapeDtypeStruct((2, NP, 128), jnp.float32),
        mesh=_mesh(),
        compiler_params=pltpu.CompilerParams(needs_layout_passes=False),
        scratch_types=[
            pltpu.VMEM((CH,), jnp.int32),
            pltpu.VMEM((CH,), jnp.int32),
            pltpu.VMEM((CH,), jnp.int32),
            pltpu.VMEM((CH,), jnp.float32),
            pltpu.VMEM((CH, 128), jnp.float32),
            pltpu.VMEM((CH, 128), jnp.float32),
            pltpu.VMEM_SHARED((NP, 128), jnp.float32),
            pltpu.SemaphoreType.DMA,
        ],
    )(tab, srcp, dstp, exv)


def kernel(x, edge_index, W1_root, W1_rel, b1, W2_root, W2_rel, b2,
           Wa, a_src, a_dst, Wfc, bfc):
    f32 = jnp.float32
    xp = jnp.zeros((NP, F), f32).at[:N].set(x)
    pad = jnp.full((EP - E,), NP - 1, jnp.int32)
    srcp = jnp.concatenate([edge_index[0].astype(jnp.int32), pad])
    dstp = jnp.concatenate([edge_index[1].astype(jnp.int32), pad])

    xt = xp.reshape(NP, 2, 128).transpose(1, 0, 2).reshape(2 * NP, 128)
    agg1 = _seg_sum(xt, srcp, dstp)
    h1t = _layer1(xp, agg1, W1_root, W1_rel, b1.reshape(1, H))
    agg2 = _seg_sum(h1t.reshape(2 * NP, 128), srcp, dstp)
    ha, asv, adv = _attn(h1t, agg2, W2_root, W2_rel, b2.reshape(1, H),
                         Wa, a_src.reshape(H, 1), a_dst.reshape(H, 1))
    ex, denom = _attn_scalar(asv.reshape(NP), adv.reshape(NP), srcp, dstp)
    gat = _gat(ha.reshape(2 * NP, 128), srcp, dstp, ex)
    out = _final(gat, denom.reshape(NP, 1), Wfc, bfc.reshape(1, O))
    return out[:N]
